# baseline (speedup 1.0000x reference)
import jax
import jax.numpy as jnp
from jax import lax
from jax.experimental import pallas as pl
from jax.experimental.pallas import tpu as pltpu
from jax.experimental.pallas import tpu_sc as plsc

NUM_COLORS = 100
NUM_SHAPES = 100
D = 128
BATCH = 16384

NC = 2
NS = 16
NW = NC * NS
BPW = BATCH // NW
CHUNK = 128
NCH = BPW // CHUNK
NBUF = 4

PAD = 128
TCB = 1024
NBLK = BATCH // TCB


def _sc_body(color_tbl, cidx, out, idx_v, tbl_v, rows_v, gsem, wsem):
    wid = lax.axis_index("s") * NC + lax.axis_index("c")
    base = wid * BPW

    pltpu.sync_copy(cidx.at[wid], idx_v)
    @pl.when(lax.axis_index("s") == 0)
    def _stage():
        pltpu.sync_copy(color_tbl, tbl_v)
    plsc.subcore_barrier()

    g = [None] * NCH
    w = [None] * NCH

    def fire_write(k):
        return pltpu.async_copy(
            rows_v.at[k % NBUF],
            out.at[0, pl.ds(base + k * CHUNK, CHUNK)],
            wsem.at[k % NBUF])

    for k in range(NCH):
        p = k % NBUF
        if k >= NBUF:
            w[k - NBUF].wait()
        g[k] = pltpu.async_copy(tbl_v.at[idx_v.at[k]], rows_v.at[p],
                                gsem.at[p])
        if k >= 1:
            g[k - 1].wait()
            w[k - 1] = fire_write(k - 1)
    g[NCH - 1].wait()
    w[NCH - 1] = fire_write(NCH - 1)
    for k in range(max(0, NCH - NBUF), NCH):
        w[k].wait()


def _tc_body(alias_ref, idx_ref, tbl_ref, out_ref):
    del alias_ref
    idxb = idx_ref[0]
    onehot_t = (lax.broadcasted_iota(jnp.int32, (PAD, TCB), 0)
                == idxb).astype(jnp.float32)
    out_ref[0] = lax.dot_general(
        onehot_t, tbl_ref[...], (((0,), (0,)), ((), ())),
        precision=lax.Precision.HIGHEST,
        preferred_element_type=jnp.float32)


def kernel(color_tbl, shape_tbl, color_idx, shape_idx):
    cidx = color_idx.reshape(NW, NCH, CHUNK)
    mesh = plsc.VectorSubcoreMesh(core_axis_name="c", subcore_axis_name="s")
    sc = pl.kernel(
        _sc_body,
        out_type=jax.ShapeDtypeStruct((2, BATCH, D), jnp.float32),
        mesh=mesh,
        scratch_types=[
            pltpu.VMEM((NCH, CHUNK), jnp.int32),
            pltpu.VMEM_SHARED((NUM_COLORS, D), jnp.float32),
            pltpu.VMEM((NBUF, CHUNK, D), jnp.float32),
            pltpu.SemaphoreType.DMA((NBUF,)),
            pltpu.SemaphoreType.DMA((NBUF,)),
        ],
    )
    half = sc(color_tbl, cidx)

    tbl_pad = jnp.zeros((PAD, D), jnp.float32).at[:NUM_SHAPES].set(shape_tbl)
    sidx = shape_idx.reshape(NBLK, 1, TCB)
    tc = pl.pallas_call(
        _tc_body,
        grid=(NBLK,),
        in_specs=[
            pl.BlockSpec(memory_space=pl.ANY),
            pl.BlockSpec((1, 1, TCB), lambda g: (g, 0, 0)),
            pl.BlockSpec((PAD, D), lambda g: (0, 0)),
        ],
        out_specs=pl.BlockSpec((1, TCB, D), lambda g: (1, g, 0)),
        out_shape=jax.ShapeDtypeStruct((2, BATCH, D), jnp.float32),
        input_output_aliases={0: 0},
    )
    return tc(half, sidx, tbl_pad)


# speedup vs baseline: 1.1961x; 1.1961x over previous
import jax
import jax.numpy as jnp
from jax import lax
from jax.experimental import pallas as pl
from jax.experimental.pallas import tpu as pltpu
from jax.experimental.pallas import tpu_sc as plsc

NUM_COLORS = 100
NUM_SHAPES = 100
D = 128
BATCH = 16384

NC = 2
NS = 16
NW = NC * NS
BPW = BATCH // NW
CHUNK = 128
NCH = BPW // CHUNK
NBUF = 4

PAD = 128
TCB = 2048
NBLK = BATCH // TCB


def _sc_body(color_tbl, cidx, out, idx_v, tbl_v, rows_v, gsem, wsem):
    wid = lax.axis_index("s") * NC + lax.axis_index("c")
    base = wid * BPW

    pltpu.sync_copy(cidx.at[wid], idx_v)
    @pl.when(lax.axis_index("s") == 0)
    def _stage():
        pltpu.sync_copy(color_tbl, tbl_v)
    plsc.subcore_barrier()

    g = [None] * NCH
    w = [None] * NCH

    def fire_write(k):
        return pltpu.async_copy(
            rows_v.at[k % NBUF],
            out.at[0, pl.ds(base + k * CHUNK, CHUNK)],
            wsem.at[k % NBUF])

    for k in range(NCH):
        p = k % NBUF
        if k >= NBUF:
            w[k - NBUF].wait()
        g[k] = pltpu.async_copy(tbl_v.at[idx_v.at[k]], rows_v.at[p],
                                gsem.at[p])
        if k >= 1:
            g[k - 1].wait()
            w[k - 1] = fire_write(k - 1)
    g[NCH - 1].wait()
    w[NCH - 1] = fire_write(NCH - 1)
    for k in range(max(0, NCH - NBUF), NCH):
        w[k].wait()


def _tc_body(alias_ref, idx_ref, tbl_ref, out_ref):
    del alias_ref
    idxb = idx_ref[0]
    onehot_t = (lax.broadcasted_iota(jnp.int32, (PAD, TCB), 0)
                == idxb).astype(jnp.float32)
    out_ref[0] = lax.dot_general(
        onehot_t, tbl_ref[...], (((0,), (0,)), ((), ())),
        preferred_element_type=jnp.float32)


def kernel(color_tbl, shape_tbl, color_idx, shape_idx):
    cidx = color_idx.reshape(NW, NCH, CHUNK)
    mesh = plsc.VectorSubcoreMesh(core_axis_name="c", subcore_axis_name="s")
    sc = pl.kernel(
        _sc_body,
        out_type=jax.ShapeDtypeStruct((2, BATCH, D), jnp.float32),
        mesh=mesh,
        scratch_types=[
            pltpu.VMEM((NCH, CHUNK), jnp.int32),
            pltpu.VMEM_SHARED((NUM_COLORS, D), jnp.float32),
            pltpu.VMEM((NBUF, CHUNK, D), jnp.float32),
            pltpu.SemaphoreType.DMA((NBUF,)),
            pltpu.SemaphoreType.DMA((NBUF,)),
        ],
    )
    half = sc(color_tbl, cidx)

    tbl_pad = jnp.zeros((PAD, D), jnp.float32).at[:NUM_SHAPES].set(shape_tbl)
    sidx = shape_idx.reshape(NBLK, 1, TCB)
    tc = pl.pallas_call(
        _tc_body,
        grid=(NBLK,),
        in_specs=[
            pl.BlockSpec(memory_space=pl.ANY),
            pl.BlockSpec((1, 1, TCB), lambda g: (g, 0, 0)),
            pl.BlockSpec((PAD, D), lambda g: (0, 0)),
        ],
        out_specs=pl.BlockSpec((1, TCB, D), lambda g: (1, g, 0)),
        out_shape=jax.ShapeDtypeStruct((2, BATCH, D), jnp.float32),
        input_output_aliases={0: 0},
    )
    return tc(half, sidx, tbl_pad)


# speedup vs baseline: 1.2874x; 1.0763x over previous
import jax
import jax.numpy as jnp
from jax import lax
from jax.experimental import pallas as pl
from jax.experimental.pallas import tpu as pltpu
from jax.experimental.pallas import tpu_sc as plsc

NUM_COLORS = 100
NUM_SHAPES = 100
D = 128
BATCH = 16384

NC = 2
NS = 16
NW = NC * NS
BPW = BATCH // NW
CHUNK = 128
NCH = BPW // CHUNK
PAIR = 2 * CHUNK
NP = 2 * BPW // PAIR


def _body(color_tbl, shape_tbl, cidx, sidx, out, idx_v, ctbl_v, stbl_v,
          rows_v, gsem, wsem):
    wid = lax.axis_index("s") * NC + lax.axis_index("c")
    base = wid * BPW

    pltpu.sync_copy(cidx.at[wid], idx_v.at[0])
    pltpu.sync_copy(sidx.at[wid], idx_v.at[1])
    @pl.when(lax.axis_index("s") == 0)
    def _stage():
        pltpu.sync_copy(color_tbl, ctbl_v)
        pltpu.sync_copy(shape_tbl, stbl_v)
    plsc.subcore_barrier()

    tbls = (ctbl_v, ctbl_v, stbl_v, stbl_v)
    g = [None] * (2 * NCH)
    w = [None] * NP

    def fire_gathers(p):
        b = p % 2
        t, half = p // 2, p % 2
        for h in range(2):
            j = 2 * half + h
            g[2 * p + h] = pltpu.async_copy(
                tbls[p].at[idx_v.at[t, j]],
                rows_v.at[b, pl.ds(h * CHUNK, CHUNK)],
                gsem.at[b, h])

    def fire_write(p):
        b = p % 2
        t, half = p // 2, p % 2
        return pltpu.async_copy(
            rows_v.at[b],
            out.at[t, pl.ds(base + half * PAIR, PAIR)],
            wsem.at[b])

    for p in range(NP):
        if p >= 2:
            w[p - 2].wait()
        fire_gathers(p)
        if p >= 1:
            g[2 * p - 2].wait()
            g[2 * p - 1].wait()
            w[p - 1] = fire_write(p - 1)
    g[2 * NP - 2].wait()
    g[2 * NP - 1].wait()
    w[NP - 1] = fire_write(NP - 1)
    w[NP - 2].wait()
    w[NP - 1].wait()


def kernel(color_tbl, shape_tbl, color_idx, shape_idx):
    cidx = color_idx.reshape(NW, NCH, CHUNK)
    sidx = shape_idx.reshape(NW, NCH, CHUNK)
    mesh = plsc.VectorSubcoreMesh(core_axis_name="c", subcore_axis_name="s")
    f = pl.kernel(
        _body,
        out_type=jax.ShapeDtypeStruct((2, BATCH, D), jnp.float32),
        mesh=mesh,
        scratch_types=[
            pltpu.VMEM((2, NCH, CHUNK), jnp.int32),
            pltpu.VMEM_SHARED((NUM_COLORS, D), jnp.float32),
            pltpu.VMEM_SHARED((NUM_SHAPES, D), jnp.float32),
            pltpu.VMEM((2, PAIR, D), jnp.float32),
            pltpu.SemaphoreType.DMA((2, 2)),
            pltpu.SemaphoreType.DMA((2,)),
        ],
    )
    return f(color_tbl, shape_tbl, cidx, sidx)


# speedup vs baseline: 1.3861x; 1.0767x over previous
import jax
import jax.numpy as jnp
from jax import lax
from jax.experimental import pallas as pl
from jax.experimental.pallas import tpu as pltpu
from jax.experimental.pallas import tpu_sc as plsc

NUM_COLORS = 100
NUM_SHAPES = 100
D = 128
BATCH = 16384

NC = 2
NS = 16
NW = NC * NS
BPW = BATCH // NW
CHUNK = 128
NCH = BPW // CHUNK
PAIR = 2 * CHUNK
NP = 2 * BPW // PAIR


NRB = 3


def _body(color_tbl, shape_tbl, cidx, sidx, out, idx_v, ctbl_v, stbl_v,
          rows_v, gsem, wsem, isem):
    wid = lax.axis_index("s") * NC + lax.axis_index("c")
    base = wid * BPW

    i0 = pltpu.async_copy(cidx.at[wid], idx_v.at[0], isem.at[0])
    i1 = pltpu.async_copy(sidx.at[wid], idx_v.at[1], isem.at[1])
    @pl.when(lax.axis_index("s") == 0)
    def _stage():
        s0 = pltpu.async_copy(color_tbl, ctbl_v, isem.at[2])
        s1 = pltpu.async_copy(shape_tbl, stbl_v, isem.at[3])
        s0.wait()
        s1.wait()
    i0.wait()
    i1.wait()
    plsc.subcore_barrier()

    tbls = (ctbl_v, ctbl_v, stbl_v, stbl_v)
    g = [None] * (2 * NCH)
    w = [None] * NP

    def fire_gathers(p):
        b = p % NRB
        t, half = p // 2, p % 2
        for h in range(2):
            j = 2 * half + h
            g[2 * p + h] = pltpu.async_copy(
                tbls[p].at[idx_v.at[t, j]],
                rows_v.at[b, pl.ds(h * CHUNK, CHUNK)],
                gsem.at[b, h])

    def fire_write(p):
        b = p % NRB
        t, half = p // 2, p % 2
        return pltpu.async_copy(
            rows_v.at[b],
            out.at[t, pl.ds(base + half * PAIR, PAIR)],
            wsem.at[b])

    for p in range(NP):
        if p >= NRB:
            w[p - NRB].wait()
        fire_gathers(p)
        if p >= 1:
            g[2 * p - 2].wait()
            g[2 * p - 1].wait()
            w[p - 1] = fire_write(p - 1)
    g[2 * NP - 2].wait()
    g[2 * NP - 1].wait()
    w[NP - 1] = fire_write(NP - 1)
    for p in range(max(0, NP - NRB), NP):
        w[p].wait()


def kernel(color_tbl, shape_tbl, color_idx, shape_idx):
    cidx = color_idx.reshape(NW, NCH, CHUNK)
    sidx = shape_idx.reshape(NW, NCH, CHUNK)
    mesh = plsc.VectorSubcoreMesh(core_axis_name="c", subcore_axis_name="s")
    f = pl.kernel(
        _body,
        out_type=jax.ShapeDtypeStruct((2, BATCH, D), jnp.float32),
        mesh=mesh,
        scratch_types=[
            pltpu.VMEM((2, NCH, CHUNK), jnp.int32),
            pltpu.VMEM_SHARED((NUM_COLORS, D), jnp.float32),
            pltpu.VMEM_SHARED((NUM_SHAPES, D), jnp.float32),
            pltpu.VMEM((NRB, PAIR, D), jnp.float32),
            pltpu.SemaphoreType.DMA((NRB, 2)),
            pltpu.SemaphoreType.DMA((NRB,)),
            pltpu.SemaphoreType.DMA((4,)),
        ],
    )
    return f(color_tbl, shape_tbl, cidx, sidx)
